# baseline (speedup 1.0000x reference)
import functools
import math

import jax
import jax.numpy as jnp
import numpy as np
from jax import lax
from jax.experimental import pallas as pl
from jax.experimental.pallas import tpu as pltpu
from jax.experimental.pallas import tpu_sc as plsc

ED = 64
CAT = 512
TIME_START, TIME_STEPS = 512, 1000
NOTE_START, NOTE_RANGE = 1512, 128
VEL_START, VEL_BINS = 1640, 128
DUR_START, DUR_STEPS = 1768, 1000
VOCAB = 2768


def _pitch_block() -> np.ndarray:
    p = np.arange(NOTE_RANGE, dtype=np.float64)
    octave = np.floor(p / 12.0)
    semitone = np.mod(p, 12.0)
    i = np.arange(ED // 4, dtype=np.float64)
    div = 2.0 ** i
    s1 = np.sin(octave[:, None] / div)
    c1 = np.cos(octave[:, None] / div)
    s2 = np.sin(semitone[:, None] * (math.pi / 6.0) * (i + 1.0))
    c2 = np.cos(semitone[:, None] * (math.pi / 6.0) * (i + 1.0))
    return np.stack([s1, c1, s2, c2], axis=-1).reshape(NOTE_RANGE, ED).astype(np.float32)


_PITCH = _pitch_block()


def _build_body(table_ref, ts_a_ref, ts_b_ref, dur_a_ref, dur_b_ref,
                vel_w1_ref, vel_b1_ref, vel_w2_ref, vel_b2_ref,
                vel_g_ref, vel_bt_ref, pitch_ref, out_ref):
    out_ref[0:CAT, :] = table_ref[:, :]

    lane_is_lin = lax.broadcasted_iota(jnp.int32, (1, ED), 1) < (ED // 2)

    def tv(nsteps, a_ref, b_ref):
        t = lax.broadcasted_iota(jnp.int32, (nsteps, 1), 0).astype(jnp.float32) / float(nsteps)
        z = t * a_ref[:, :] + b_ref[:, :]
        return jnp.where(lane_is_lin, z, jnp.sin(z))

    out_ref[TIME_START:TIME_START + TIME_STEPS, :] = tv(TIME_STEPS, ts_a_ref, ts_b_ref)
    out_ref[DUR_START:DUR_START + DUR_STEPS, :] = tv(DUR_STEPS, dur_a_ref, dur_b_ref)

    out_ref[NOTE_START:NOTE_START + NOTE_RANGE, :] = pitch_ref[:, :]

    v = lax.broadcasted_iota(jnp.int32, (VEL_BINS, 1), 0).astype(jnp.float32) / float(VEL_BINS)
    h = v * vel_w1_ref[:, :] + vel_b1_ref[:, :]
    h = 0.5 * h * (1.0 + lax.erf(h * (1.0 / math.sqrt(2.0))))
    o = jnp.dot(h, vel_w2_ref[:, :], preferred_element_type=jnp.float32) + vel_b2_ref[:, :]
    mu = jnp.mean(o, axis=-1, keepdims=True)
    var = jnp.mean((o - mu) ** 2, axis=-1, keepdims=True)
    o = (o - mu) / jnp.sqrt(var + 1e-5) * vel_g_ref[:, :] + vel_bt_ref[:, :]
    out_ref[VEL_START:VEL_START + VEL_BINS, :] = o


_build_table = pl.pallas_call(
    _build_body,
    out_shape=jax.ShapeDtypeStruct((VOCAB, ED), jnp.float32),
)


NC, NS = 2, 16
NW = NC * NS
SEQ, TOK = 1024, 200
B_TOK = SEQ * TOK
BPW = B_TOK // NW
CH = 800
NCH = BPW // CH


@functools.cache
def _make_gather():
    mesh = plsc.VectorSubcoreMesh(core_axis_name="c", subcore_axis_name="s")

    @functools.partial(
        pl.kernel,
        mesh=mesh,
        compiler_params=pltpu.CompilerParams(use_tc_tiling_on_sc=False),
        out_type=jax.ShapeDtypeStruct((B_TOK, 2 * ED), jnp.float32),
        scratch_types=[
            pltpu.VMEM((BPW,), jnp.int32),
            pltpu.VMEM((CH, ED), jnp.float32),
            pltpu.VMEM((CH, ED), jnp.float32),
            pltpu.SemaphoreType.DMA,
            pltpu.SemaphoreType.DMA,
            pltpu.SemaphoreType.DMA,
            pltpu.SemaphoreType.DMA,
        ],
    )
    def _gather_rows(table_hbm, idx_hbm, out_hbm, idx_v, buf0, buf1,
                     gsem0, gsem1, wsem0, wsem1):
        wid = lax.axis_index("s") * NC + lax.axis_index("c")
        base = wid * BPW
        pltpu.sync_copy(idx_hbm.at[pl.ds(base, BPW)], idx_v)

        bufs = (buf0, buf1)
        gsems = (gsem0, gsem1)
        wsems = (wsem0, wsem1)

        def start_gather(c, b):
            return pltpu.async_copy(
                table_hbm.at[idx_v.at[pl.ds(c * CH, CH)]], bufs[b], gsems[b])

        gathers = [start_gather(0, 0), start_gather(1, 1)]
        writes = [None, None]
        for c in range(NCH):
            b = c % 2
            gathers[b].wait()
            writes[b] = pltpu.async_copy(
                bufs[b], out_hbm.at[pl.ds(base + c * CH, CH), pl.ds(0, ED)],
                wsems[b])
            if c + 2 < NCH:
                writes[b].wait()
                gathers[b] = start_gather(c + 2, b)
        for c in range(NCH - 2, NCH):
            writes[c % 2].wait()

    return _gather_rows


def _relayout_body(in_ref, out_ref):
    out_ref[...] = in_ref[:, 0:ED].reshape(2, TOK, ED)


_relayout = pl.pallas_call(
    _relayout_body,
    grid=(SEQ // 2,),
    in_specs=[pl.BlockSpec((2 * TOK, 2 * ED), lambda i: (i, 0))],
    out_specs=pl.BlockSpec((2, TOK, ED), lambda i: (i, 0, 0)),
    out_shape=jax.ShapeDtypeStruct((SEQ, TOK, ED), jnp.float32),
)


def kernel(tokens, table, ts_Wl, ts_bl, ts_w, ts_bp, vel_W1, vel_b1, vel_W2,
           vel_b2, vel_gamma, vel_beta, dur_Wl, dur_bl, dur_w, dur_bp):
    ts_a = jnp.concatenate([ts_Wl, ts_w]).reshape(1, ED)
    ts_b = jnp.concatenate([ts_bl, ts_bp]).reshape(1, ED)
    dur_a = jnp.concatenate([dur_Wl, dur_w]).reshape(1, ED)
    dur_b = jnp.concatenate([dur_bl, dur_bp]).reshape(1, ED)
    big = _build_table(
        table, ts_a, ts_b, dur_a, dur_b,
        vel_W1.reshape(1, ED // 2), vel_b1.reshape(1, ED // 2),
        vel_W2, vel_b2.reshape(1, ED),
        vel_gamma.reshape(1, ED), vel_beta.reshape(1, ED),
        jnp.asarray(_PITCH),
    )
    idx = tokens.reshape(-1).astype(jnp.int32)
    rows = _make_gather()(big, idx)
    return _relayout(rows)


# speedup vs baseline: 3.4552x; 3.4552x over previous
import functools
import math

import jax
import jax.numpy as jnp
import numpy as np
from jax import lax
from jax.experimental import pallas as pl
from jax.experimental.pallas import tpu as pltpu
from jax.experimental.pallas import tpu_sc as plsc

ED = 64
CAT = 512
TIME_START, TIME_STEPS = 512, 1000
NOTE_START, NOTE_RANGE = 1512, 128
VEL_START, VEL_BINS = 1640, 128
DUR_START, DUR_STEPS = 1768, 1000
VOCAB = 2768


def _pitch_block() -> np.ndarray:
    p = np.arange(NOTE_RANGE, dtype=np.float64)
    octave = np.floor(p / 12.0)
    semitone = np.mod(p, 12.0)
    i = np.arange(ED // 4, dtype=np.float64)
    div = 2.0 ** i
    s1 = np.sin(octave[:, None] / div)
    c1 = np.cos(octave[:, None] / div)
    s2 = np.sin(semitone[:, None] * (math.pi / 6.0) * (i + 1.0))
    c2 = np.cos(semitone[:, None] * (math.pi / 6.0) * (i + 1.0))
    return np.stack([s1, c1, s2, c2], axis=-1).reshape(NOTE_RANGE, ED).astype(np.float32)


_PITCH = _pitch_block()


def _build_body(table_ref, ts_a_ref, ts_b_ref, dur_a_ref, dur_b_ref,
                vel_w1_ref, vel_b1_ref, vel_w2_ref, vel_b2_ref,
                vel_g_ref, vel_bt_ref, pitch_ref, out_ref):
    out_ref[0:CAT, :] = table_ref[:, :]

    lane_is_lin = lax.broadcasted_iota(jnp.int32, (1, ED), 1) < (ED // 2)

    def tv(nsteps, a_ref, b_ref):
        t = lax.broadcasted_iota(jnp.int32, (nsteps, 1), 0).astype(jnp.float32) / float(nsteps)
        z = t * a_ref[:, :] + b_ref[:, :]
        return jnp.where(lane_is_lin, z, jnp.sin(z))

    out_ref[TIME_START:TIME_START + TIME_STEPS, :] = tv(TIME_STEPS, ts_a_ref, ts_b_ref)
    out_ref[DUR_START:DUR_START + DUR_STEPS, :] = tv(DUR_STEPS, dur_a_ref, dur_b_ref)

    out_ref[NOTE_START:NOTE_START + NOTE_RANGE, :] = pitch_ref[:, :]

    v = lax.broadcasted_iota(jnp.int32, (VEL_BINS, 1), 0).astype(jnp.float32) / float(VEL_BINS)
    h = v * vel_w1_ref[:, :] + vel_b1_ref[:, :]
    h = 0.5 * h * (1.0 + lax.erf(h * (1.0 / math.sqrt(2.0))))
    o = jnp.dot(h, vel_w2_ref[:, :], preferred_element_type=jnp.float32) + vel_b2_ref[:, :]
    mu = jnp.mean(o, axis=-1, keepdims=True)
    var = jnp.mean((o - mu) ** 2, axis=-1, keepdims=True)
    o = (o - mu) / jnp.sqrt(var + 1e-5) * vel_g_ref[:, :] + vel_bt_ref[:, :]
    out_ref[VEL_START:VEL_START + VEL_BINS, :] = o


_build_table = pl.pallas_call(
    _build_body,
    out_shape=jax.ShapeDtypeStruct((VOCAB, ED), jnp.float32),
)


NC, NS = 2, 16
NW = NC * NS
SEQ, TOK = 1024, 200
B_TOK = SEQ * TOK
BPW = B_TOK // NW
CH = 800
NCH = BPW // CH


@functools.cache
def _make_gather():
    mesh = plsc.VectorSubcoreMesh(core_axis_name="c", subcore_axis_name="s")

    @functools.partial(
        pl.kernel,
        mesh=mesh,
        compiler_params=pltpu.CompilerParams(use_tc_tiling_on_sc=False),
        out_type=jax.ShapeDtypeStruct((B_TOK, 2 * ED), jnp.float32),
        scratch_types=[
            pltpu.VMEM((BPW,), jnp.int32),
            pltpu.VMEM((CH, ED), jnp.float32),
            pltpu.VMEM((CH, ED), jnp.float32),
            pltpu.SemaphoreType.DMA,
            pltpu.SemaphoreType.DMA,
            pltpu.SemaphoreType.DMA,
            pltpu.SemaphoreType.DMA,
        ],
    )
    def _gather_rows(table_hbm, idx_hbm, out_hbm, idx_v, buf0, buf1,
                     gsem0, gsem1, wsem0, wsem1):
        wid = lax.axis_index("s") * NC + lax.axis_index("c")
        base = wid * BPW
        pltpu.sync_copy(idx_hbm.at[pl.ds(base, BPW)], idx_v)

        bufs = (buf0, buf1)
        gsems = (gsem0, gsem1)
        wsems = (wsem0, wsem1)

        def start_gather(c, b):
            return pltpu.async_copy(
                table_hbm.at[idx_v.at[pl.ds(c * CH, CH)]], bufs[b], gsems[b])

        gathers = [start_gather(0, 0), start_gather(1, 1)]
        writes = [None, None]
        for c in range(NCH):
            b = c % 2
            gathers[b].wait()
            writes[b] = pltpu.async_copy(
                bufs[b], out_hbm.at[pl.ds(base + c * CH, CH), pl.ds(0, ED)],
                wsems[b])
            if c + 2 < NCH:
                writes[b].wait()
                gathers[b] = start_gather(c + 2, b)
        for c in range(NCH - 2, NCH):
            writes[c % 2].wait()

    return _gather_rows


def _relayout_body(in_ref, out_ref):
    out_ref[...] = in_ref[:, 0:ED].reshape(2, TOK, ED)


_relayout = pl.pallas_call(
    _relayout_body,
    grid=(SEQ // 2,),
    in_specs=[pl.BlockSpec((2 * TOK, 2 * ED), lambda i: (i, 0))],
    out_specs=pl.BlockSpec((2, TOK, ED), lambda i: (i, 0, 0)),
    out_shape=jax.ShapeDtypeStruct((SEQ, TOK, ED), jnp.float32),
)


def kernel(tokens, table, ts_Wl, ts_bl, ts_w, ts_bp, vel_W1, vel_b1, vel_W2,
           vel_b2, vel_gamma, vel_beta, dur_Wl, dur_bl, dur_w, dur_bp):
    ts_a = jnp.concatenate([ts_Wl, ts_w]).reshape(1, ED)
    ts_b = jnp.concatenate([ts_bl, ts_bp]).reshape(1, ED)
    dur_a = jnp.concatenate([dur_Wl, dur_w]).reshape(1, ED)
    dur_b = jnp.concatenate([dur_bl, dur_bp]).reshape(1, ED)
    big = _build_table(
        table, ts_a, ts_b, dur_a, dur_b,
        vel_W1.reshape(1, ED // 2), vel_b1.reshape(1, ED // 2),
        vel_W2, vel_b2.reshape(1, ED),
        vel_gamma.reshape(1, ED), vel_beta.reshape(1, ED),
        jnp.asarray(_PITCH),
    )
    idx = tokens.reshape(-1).astype(jnp.int32)
    rows = _make_gather()(big, idx)
    return rows[:, :ED].reshape(SEQ, TOK, ED)
